# Initial kernel scaffold; baseline (speedup 1.0000x reference)
import functools

import jax
import jax.numpy as jnp
from jax import lax
from jax.experimental import pallas as pl
from jax.experimental.pallas import tpu as pltpu
from jax.experimental.pallas import tpu_sc as plsc

N_USERS = 5000
N_ITEMS = 5000
N = N_USERS + N_ITEMS
D = 256
E = 160000
NUM_LAYERS = 3

NC = 2
NS = 16
LANES = 16

ROWS_PER_SC = N // NC
ROWS_PER_TILE = 313
ACC_ROWS = 5024
ZERO_ROWS = ACC_ROWS // NS
E_PER_TILE = E // NS
CHUNK = 80
N_CHUNKS = E_PER_TILE // CHUNK


def _spmm_body(x_hbm, src_hbm, dst_hbm, w_hbm, y_hbm,
               acc, rows, zbuf, src_i, dst_i, w_v, ldst, sem):
    c = lax.axis_index("c")
    s = lax.axis_index("s")
    base_row = c * ROWS_PER_SC

    zeros16 = jnp.zeros((LANES,), jnp.float32)

    def zrow(r, carry):
        for j in range(D // LANES):
            zbuf[r, pl.ds(j * LANES, LANES)] = zeros16
        return carry

    lax.fori_loop(0, ZERO_ROWS, zrow, 0)
    pltpu.sync_copy(zbuf, acc.at[pl.ds(s * ZERO_ROWS, ZERO_ROWS)])
    plsc.subcore_barrier()

    e_base = s * E_PER_TILE

    def chunk_body(k, carry):
        e0 = e_base + k * CHUNK
        pltpu.sync_copy(src_hbm.at[pl.ds(e0, CHUNK)], src_i)
        pltpu.sync_copy(dst_hbm.at[pl.ds(e0, CHUNK)], dst_i)
        pltpu.sync_copy(w_hbm.at[pl.ds(e0, CHUNK)], w_v)
        pltpu.async_copy(x_hbm.at[src_i], rows, sem).wait()

        for g in range(CHUNK // LANES):
            dv = dst_i[pl.ds(g * LANES, LANES)]
            wv = w_v[pl.ds(g * LANES, LANES)]
            rel = dv - base_row
            ok = (rel >= 0) & (rel < ROWS_PER_SC)
            ldst[pl.ds(g * LANES, LANES)] = jnp.clip(rel, 0, ROWS_PER_SC - 1)
            w_v[pl.ds(g * LANES, LANES)] = jnp.where(ok, wv, 0.0)

        def scale_edge(e, carry2):
            bw = plsc.load_gather(w_v, [jnp.full((LANES,), e, jnp.int32)])
            for j in range(D // LANES):
                rows[e, pl.ds(j * LANES, LANES)] = (
                    rows[e, pl.ds(j * LANES, LANES)] * bw)
            return carry2

        lax.fori_loop(0, CHUNK, scale_edge, 0)

        pltpu.sync_copy(rows, acc.at[ldst], add=True)
        return carry

    lax.fori_loop(0, N_CHUNKS, chunk_body, 0)
    plsc.subcore_barrier()

    r0 = jnp.minimum(s * ROWS_PER_TILE, ROWS_PER_SC - ROWS_PER_TILE)
    pltpu.sync_copy(acc.at[pl.ds(r0, ROWS_PER_TILE)],
                    y_hbm.at[pl.ds(base_row + r0, ROWS_PER_TILE)])


_spmm = functools.partial(
    pl.kernel,
    out_type=jax.ShapeDtypeStruct((N, D), jnp.float32),
    mesh=plsc.VectorSubcoreMesh(core_axis_name="c", subcore_axis_name="s"),
    scratch_types=[
        pltpu.VMEM_SHARED((ACC_ROWS, D), jnp.float32),
        pltpu.VMEM((CHUNK, D), jnp.float32),
        pltpu.VMEM((ZERO_ROWS, D), jnp.float32),
        pltpu.VMEM((CHUNK,), jnp.int32),
        pltpu.VMEM((CHUNK,), jnp.int32),
        pltpu.VMEM((CHUNK,), jnp.float32),
        pltpu.VMEM((CHUNK,), jnp.int32),
        pltpu.SemaphoreType.DMA,
    ],
)(_spmm_body)


LAMBDAS = [float(l + 1) / (NUM_LAYERS + 1) for l in range(NUM_LAYERS + 1)]
BIN_BLOCK = 1000


def _binarize_body(x0, x1, x2, x3, o):
    for i, (xr, lam) in enumerate(zip((x0, x1, x2, x3), LAMBDAS)):
        v = xr[...]
        m = jnp.mean(jnp.abs(v), axis=1, keepdims=True)
        o[:, i, :] = jnp.sign(v) * (m * lam)


def _binarize4(xs):
    in_spec = pl.BlockSpec((BIN_BLOCK, D), lambda i: (i, 0))
    return pl.pallas_call(
        _binarize_body,
        grid=(N // BIN_BLOCK,),
        in_specs=[in_spec] * 4,
        out_specs=pl.BlockSpec((BIN_BLOCK, NUM_LAYERS + 1, D),
                               lambda i: (i, 0, 0)),
        out_shape=jax.ShapeDtypeStruct((N, NUM_LAYERS + 1, D), jnp.float32),
    )(*xs)


def kernel(user_embed, item_embed, edge_index, edge_weight):
    con = jnp.concatenate([user_embed, item_embed], axis=0)
    src = edge_index[0]
    dst = edge_index[1]
    xs = [con]
    for _ in range(NUM_LAYERS):
        xs.append(_spmm(xs[-1], src, dst, edge_weight))
    return _binarize4(xs)


# speedup vs baseline: 2.4340x; 2.4340x over previous
import functools

import jax
import jax.numpy as jnp
from jax import lax
from jax.experimental import pallas as pl
from jax.experimental.pallas import tpu as pltpu
from jax.experimental.pallas import tpu_sc as plsc

N_USERS = 5000
N_ITEMS = 5000
N = N_USERS + N_ITEMS
D = 256
E = 160000
NUM_LAYERS = 3

NC = 2
NS = 16
LANES = 16
HCOL = D // NC

NP_ = 10112
ROWS_PER_TILE = NP_ // NS
E_PER_TILE = E // NS
CHUNK = 80
N_CHUNKS = E_PER_TILE // CHUNK


def _spmm_body(x_hbm, src_hbm, dst_hbm, w_hbm, z_hbm, y_hbm,
               acc, rows, src_i, gsrc, dst_i, w_v, sem):
    c = lax.axis_index("c")
    s = lax.axis_index("s")

    r0 = pl.multiple_of(s * ROWS_PER_TILE, 8)
    pltpu.sync_copy(z_hbm.at[pl.ds(r0, ROWS_PER_TILE)],
                    acc.at[pl.ds(r0, ROWS_PER_TILE)])
    plsc.subcore_barrier()

    e_base = s * E_PER_TILE

    def chunk_body(k, carry):
        e0 = pl.multiple_of(e_base + k * CHUNK, 8)
        pltpu.sync_copy(src_hbm.at[pl.ds(e0, CHUNK)], src_i)
        pltpu.sync_copy(dst_hbm.at[pl.ds(e0, CHUNK)], dst_i)
        pltpu.sync_copy(w_hbm.at[pl.ds(e0, CHUNK)], w_v)

        for g in range(CHUNK // LANES):
            sv = src_i[pl.ds(g * LANES, LANES)]
            gsrc[pl.ds(g * LANES, LANES)] = sv + c * NP_

        pltpu.async_copy(x_hbm.at[gsrc], rows, sem).wait()

        def scale_edge(e, carry2):
            bw = plsc.load_gather(w_v, [jnp.full((LANES,), e, jnp.int32)])
            for j in range(HCOL // LANES):
                rows[e, pl.ds(j * LANES, LANES)] = (
                    rows[e, pl.ds(j * LANES, LANES)] * bw)
            return carry2

        lax.fori_loop(0, CHUNK, scale_edge, 0)

        pltpu.sync_copy(rows, acc.at[dst_i], add=True)
        return carry

    lax.fori_loop(0, N_CHUNKS, chunk_body, 0)
    plsc.subcore_barrier()

    y0 = pl.multiple_of(c * NP_ + s * ROWS_PER_TILE, 8)
    pltpu.sync_copy(acc.at[pl.ds(r0, ROWS_PER_TILE)],
                    y_hbm.at[pl.ds(y0, ROWS_PER_TILE)])


_spmm_call = functools.partial(
    pl.kernel,
    out_type=jax.ShapeDtypeStruct((NC * NP_, HCOL), jnp.float32),
    mesh=plsc.VectorSubcoreMesh(core_axis_name="c", subcore_axis_name="s"),
    compiler_params=pltpu.CompilerParams(needs_layout_passes=False),
    scratch_types=[
        pltpu.VMEM_SHARED((NP_, HCOL), jnp.float32),
        pltpu.VMEM((CHUNK, HCOL), jnp.float32),
        pltpu.VMEM((CHUNK,), jnp.int32),
        pltpu.VMEM((CHUNK,), jnp.int32),
        pltpu.VMEM((CHUNK,), jnp.int32),
        pltpu.VMEM((CHUNK,), jnp.float32),
        pltpu.SemaphoreType.DMA,
    ],
)(_spmm_body)


def _to_half(x):
    pad = NP_ - N
    return jnp.concatenate(
        [jnp.pad(x[:, :HCOL], ((0, pad), (0, 0))),
         jnp.pad(x[:, HCOL:], ((0, pad), (0, 0)))], axis=0)


def _from_half(h):
    return jnp.concatenate([h[:N], h[NP_:NP_ + N]], axis=1)


LAMBDAS = [float(l + 1) / (NUM_LAYERS + 1) for l in range(NUM_LAYERS + 1)]
BIN_BLOCK = 1000


def _binarize_body(x0, x1, x2, x3, o):
    for i, (xr, lam) in enumerate(zip((x0, x1, x2, x3), LAMBDAS)):
        v = xr[...]
        m = jnp.mean(jnp.abs(v), axis=1, keepdims=True)
        o[:, i, :] = jnp.sign(v) * (m * lam)


def _binarize4(xs):
    in_spec = pl.BlockSpec((BIN_BLOCK, D), lambda i: (i, 0))
    return pl.pallas_call(
        _binarize_body,
        grid=(N // BIN_BLOCK,),
        in_specs=[in_spec] * 4,
        out_specs=pl.BlockSpec((BIN_BLOCK, NUM_LAYERS + 1, D),
                               lambda i: (i, 0, 0)),
        out_shape=jax.ShapeDtypeStruct((N, NUM_LAYERS + 1, D), jnp.float32),
    )(*xs)


def kernel(user_embed, item_embed, edge_index, edge_weight):
    con = jnp.concatenate([user_embed, item_embed], axis=0)
    src = edge_index[0]
    dst = edge_index[1]
    zeros = jnp.zeros((NP_, HCOL), jnp.float32)
    xs = [con]
    h = _to_half(con)
    for _ in range(NUM_LAYERS):
        h = _spmm_call(h, src, dst, edge_weight, zeros)
        xs.append(_from_half(h))
    return _binarize4(xs)


# speedup vs baseline: 3.2994x; 1.3556x over previous
import functools

import jax
import jax.numpy as jnp
from jax import lax
from jax.experimental import pallas as pl
from jax.experimental.pallas import tpu as pltpu
from jax.experimental.pallas import tpu_sc as plsc

N_USERS = 5000
N_ITEMS = 5000
N = N_USERS + N_ITEMS
D = 256
E = 160000
NUM_LAYERS = 3

NC = 2
NS = 16
LANES = 16
HCOL = D // NC

NP_ = 10112
ROWS_PER_TILE = NP_ // NS
E_PER_TILE = E // NS
CHUNK = 80
N_CHUNKS = E_PER_TILE // CHUNK


def _spmm_body(x_hbm, src_hbm, dst_hbm, w_hbm, z_hbm, y_hbm,
               acc, rows0, rows1, src_i0, src_i1, gsrc0, gsrc1,
               dst_i0, dst_i1, w_v0, w_v1, sem0, sem1):
    c = lax.axis_index("c")
    s = lax.axis_index("s")
    rows = (rows0, rows1)
    src_i = (src_i0, src_i1)
    gsrc = (gsrc0, gsrc1)
    dst_i = (dst_i0, dst_i1)
    w_v = (w_v0, w_v1)
    sem = (sem0, sem1)

    r0 = pl.multiple_of(s * ROWS_PER_TILE, 8)
    pltpu.sync_copy(z_hbm.at[pl.ds(r0, ROWS_PER_TILE)],
                    acc.at[pl.ds(r0, ROWS_PER_TILE)])
    plsc.subcore_barrier()

    e_base = s * E_PER_TILE

    def prep(k, b):
        e0 = pl.multiple_of(e_base + k * CHUNK, 8)
        pltpu.sync_copy(src_hbm.at[pl.ds(e0, CHUNK)], src_i[b])
        pltpu.sync_copy(dst_hbm.at[pl.ds(e0, CHUNK)], dst_i[b])
        pltpu.sync_copy(w_hbm.at[pl.ds(e0, CHUNK)], w_v[b])
        for g in range(CHUNK // LANES):
            sv = src_i[b][pl.ds(g * LANES, LANES)]
            gsrc[b][pl.ds(g * LANES, LANES)] = sv + c * NP_
        return pltpu.async_copy(x_hbm.at[gsrc[b]], rows[b], sem[b])

    def consume(b):
        def scale_edge(e, carry2):
            bw = plsc.load_gather(w_v[b], [jnp.full((LANES,), e, jnp.int32)])
            for j in range(HCOL // LANES):
                rows[b][e, pl.ds(j * LANES, LANES)] = (
                    rows[b][e, pl.ds(j * LANES, LANES)] * bw)
            return carry2

        lax.fori_loop(0, CHUNK, scale_edge, 0)
        pltpu.sync_copy(rows[b], acc.at[dst_i[b]], add=True)

    def wait_g(b):
        pltpu.make_async_copy(x_hbm.at[gsrc[b]], rows[b], sem[b]).wait()

    prep(0, 0)

    def pair_body(p, carry):
        k = 2 * p
        prep(k + 1, 1)
        wait_g(0)
        consume(0)
        prep(k + 2, 0)
        wait_g(1)
        consume(1)
        return carry

    lax.fori_loop(0, (N_CHUNKS - 1) // 2, pair_body, 0)
    wait_g(0)
    consume(0)
    plsc.subcore_barrier()

    y0 = pl.multiple_of(c * NP_ + s * ROWS_PER_TILE, 8)
    pltpu.sync_copy(acc.at[pl.ds(r0, ROWS_PER_TILE)],
                    y_hbm.at[pl.ds(y0, ROWS_PER_TILE)])


_spmm_call = functools.partial(
    pl.kernel,
    out_type=jax.ShapeDtypeStruct((NC * NP_, HCOL), jnp.float32),
    mesh=plsc.VectorSubcoreMesh(core_axis_name="c", subcore_axis_name="s"),
    compiler_params=pltpu.CompilerParams(needs_layout_passes=False),
    scratch_types=[
        pltpu.VMEM_SHARED((NP_, HCOL), jnp.float32),
        pltpu.VMEM((CHUNK, HCOL), jnp.float32),
        pltpu.VMEM((CHUNK, HCOL), jnp.float32),
        pltpu.VMEM((CHUNK,), jnp.int32),
        pltpu.VMEM((CHUNK,), jnp.int32),
        pltpu.VMEM((CHUNK,), jnp.int32),
        pltpu.VMEM((CHUNK,), jnp.int32),
        pltpu.VMEM((CHUNK,), jnp.int32),
        pltpu.VMEM((CHUNK,), jnp.int32),
        pltpu.VMEM((CHUNK,), jnp.float32),
        pltpu.VMEM((CHUNK,), jnp.float32),
        pltpu.SemaphoreType.DMA,
        pltpu.SemaphoreType.DMA,
    ],
)(_spmm_body)


def _to_half(x):
    pad = NP_ - N
    return jnp.concatenate(
        [jnp.pad(x[:, :HCOL], ((0, pad), (0, 0))),
         jnp.pad(x[:, HCOL:], ((0, pad), (0, 0)))], axis=0)


def _from_half(h):
    return jnp.concatenate([h[:N], h[NP_:NP_ + N]], axis=1)


LAMBDAS = [float(l + 1) / (NUM_LAYERS + 1) for l in range(NUM_LAYERS + 1)]
BIN_BLOCK = 1000


def _binarize_body(x0, x1, x2, x3, o):
    for i, (xr, lam) in enumerate(zip((x0, x1, x2, x3), LAMBDAS)):
        v = xr[...]
        m = jnp.mean(jnp.abs(v), axis=1, keepdims=True)
        o[:, i, :] = jnp.sign(v) * (m * lam)


def _binarize4(xs):
    in_spec = pl.BlockSpec((BIN_BLOCK, D), lambda i: (i, 0))
    return pl.pallas_call(
        _binarize_body,
        grid=(N // BIN_BLOCK,),
        in_specs=[in_spec] * 4,
        out_specs=pl.BlockSpec((BIN_BLOCK, NUM_LAYERS + 1, D),
                               lambda i: (i, 0, 0)),
        out_shape=jax.ShapeDtypeStruct((N, NUM_LAYERS + 1, D), jnp.float32),
    )(*xs)


def kernel(user_embed, item_embed, edge_index, edge_weight):
    con = jnp.concatenate([user_embed, item_embed], axis=0)
    src = edge_index[0]
    dst = edge_index[1]
    zeros = jnp.zeros((NP_, HCOL), jnp.float32)
    xs = [con]
    h = _to_half(con)
    for _ in range(NUM_LAYERS):
        h = _spmm_call(h, src, dst, edge_weight, zeros)
        xs.append(_from_half(h))
    return _binarize4(xs)


# speedup vs baseline: 3.6449x; 1.1047x over previous
import functools

import jax
import jax.numpy as jnp
from jax import lax
from jax.experimental import pallas as pl
from jax.experimental.pallas import tpu as pltpu
from jax.experimental.pallas import tpu_sc as plsc

N_USERS = 5000
N_ITEMS = 5000
N = N_USERS + N_ITEMS
D = 256
E = 160000
NUM_LAYERS = 3

NC = 2
NS = 16
LANES = 16
HCOL = D // NC

NP_ = 10112
ROWS_PER_TILE = NP_ // NS
E_PER_TILE = E // NS
CHUNK = 80
N_CHUNKS = E_PER_TILE // CHUNK


def _spmm_body(x_hbm, src_hbm, dst_hbm, w_hbm, z_hbm, y_hbm,
               acc, rows0, rows1, src_i0, src_i1, gsrc0, gsrc1,
               dst_i0, dst_i1, w_v0, w_v1, sem0, sem1, sem_s0, sem_s1):
    c = lax.axis_index("c")
    s = lax.axis_index("s")
    rows = (rows0, rows1)
    src_i = (src_i0, src_i1)
    gsrc = (gsrc0, gsrc1)
    dst_i = (dst_i0, dst_i1)
    w_v = (w_v0, w_v1)
    sem = (sem0, sem1)

    r0 = pl.multiple_of(s * ROWS_PER_TILE, 8)
    pltpu.sync_copy(z_hbm.at[pl.ds(r0, ROWS_PER_TILE)],
                    acc.at[pl.ds(r0, ROWS_PER_TILE)])
    plsc.subcore_barrier()

    e_base = s * E_PER_TILE

    def prep(k, b):
        e0 = pl.multiple_of(e_base + k * CHUNK, 8)
        pltpu.sync_copy(src_hbm.at[pl.ds(e0, CHUNK)], src_i[b])
        pltpu.sync_copy(dst_hbm.at[pl.ds(e0, CHUNK)], dst_i[b])
        pltpu.sync_copy(w_hbm.at[pl.ds(e0, CHUNK)], w_v[b])
        for g in range(CHUNK // LANES):
            sv = src_i[b][pl.ds(g * LANES, LANES)]
            gsrc[b][pl.ds(g * LANES, LANES)] = sv + c * NP_
        return pltpu.async_copy(x_hbm.at[gsrc[b]], rows[b], sem[b])

    def scale(b):
        def scale_edge(e, carry2):
            bw = plsc.load_gather(w_v[b], [jnp.full((LANES,), e, jnp.int32)])
            for j in range(HCOL // LANES):
                rows[b][e, pl.ds(j * LANES, LANES)] = (
                    rows[b][e, pl.ds(j * LANES, LANES)] * bw)
            return carry2

        lax.fori_loop(0, CHUNK, scale_edge, 0)

    def wait_g(b):
        pltpu.make_async_copy(x_hbm.at[gsrc[b]], rows[b], sem[b]).wait()

    prep(0, 0)

    def pair_body(p, carry):
        k = 2 * p
        prep(k + 1, 1)
        wait_g(0)
        scale(0)
        h0 = pltpu.async_copy(rows[0], acc.at[dst_i[0]], sem_s0, add=True)
        wait_g(1)
        scale(1)
        h1 = pltpu.async_copy(rows[1], acc.at[dst_i[1]], sem_s1, add=True)
        h0.wait()
        prep(k + 2, 0)
        h1.wait()
        return carry

    lax.fori_loop(0, (N_CHUNKS - 1) // 2, pair_body, 0)
    wait_g(0)
    scale(0)
    pltpu.sync_copy(rows[0], acc.at[dst_i[0]], add=True)
    plsc.subcore_barrier()

    y0 = pl.multiple_of(c * NP_ + s * ROWS_PER_TILE, 8)
    pltpu.sync_copy(acc.at[pl.ds(r0, ROWS_PER_TILE)],
                    y_hbm.at[pl.ds(y0, ROWS_PER_TILE)])


_spmm_call = functools.partial(
    pl.kernel,
    out_type=jax.ShapeDtypeStruct((NC * NP_, HCOL), jnp.float32),
    mesh=plsc.VectorSubcoreMesh(core_axis_name="c", subcore_axis_name="s"),
    compiler_params=pltpu.CompilerParams(needs_layout_passes=False),
    scratch_types=[
        pltpu.VMEM_SHARED((NP_, HCOL), jnp.float32),
        pltpu.VMEM((CHUNK, HCOL), jnp.float32),
        pltpu.VMEM((CHUNK, HCOL), jnp.float32),
        pltpu.VMEM((CHUNK,), jnp.int32),
        pltpu.VMEM((CHUNK,), jnp.int32),
        pltpu.VMEM((CHUNK,), jnp.int32),
        pltpu.VMEM((CHUNK,), jnp.int32),
        pltpu.VMEM((CHUNK,), jnp.int32),
        pltpu.VMEM((CHUNK,), jnp.int32),
        pltpu.VMEM((CHUNK,), jnp.float32),
        pltpu.VMEM((CHUNK,), jnp.float32),
        pltpu.SemaphoreType.DMA,
        pltpu.SemaphoreType.DMA,
        pltpu.SemaphoreType.DMA,
        pltpu.SemaphoreType.DMA,
    ],
)(_spmm_body)


def _to_half(x):
    pad = NP_ - N
    return jnp.concatenate(
        [jnp.pad(x[:, :HCOL], ((0, pad), (0, 0))),
         jnp.pad(x[:, HCOL:], ((0, pad), (0, 0)))], axis=0)


def _from_half(h):
    return jnp.concatenate([h[:N], h[NP_:NP_ + N]], axis=1)


LAMBDAS = [float(l + 1) / (NUM_LAYERS + 1) for l in range(NUM_LAYERS + 1)]
BIN_BLOCK = 1000


def _binarize_body(x0, x1, x2, x3, o):
    for i, (xr, lam) in enumerate(zip((x0, x1, x2, x3), LAMBDAS)):
        v = xr[...]
        m = jnp.mean(jnp.abs(v), axis=1, keepdims=True)
        o[:, i, :] = jnp.sign(v) * (m * lam)


def _binarize4(xs):
    in_spec = pl.BlockSpec((BIN_BLOCK, D), lambda i: (i, 0))
    return pl.pallas_call(
        _binarize_body,
        grid=(N // BIN_BLOCK,),
        in_specs=[in_spec] * 4,
        out_specs=pl.BlockSpec((BIN_BLOCK, NUM_LAYERS + 1, D),
                               lambda i: (i, 0, 0)),
        out_shape=jax.ShapeDtypeStruct((N, NUM_LAYERS + 1, D), jnp.float32),
    )(*xs)


def kernel(user_embed, item_embed, edge_index, edge_weight):
    con = jnp.concatenate([user_embed, item_embed], axis=0)
    src = edge_index[0]
    dst = edge_index[1]
    zeros = jnp.zeros((NP_, HCOL), jnp.float32)
    xs = [con]
    h = _to_half(con)
    for _ in range(NUM_LAYERS):
        h = _spmm_call(h, src, dst, edge_weight, zeros)
        xs.append(_from_half(h))
    return _binarize4(xs)


# speedup vs baseline: 3.8222x; 1.0486x over previous
import functools

import jax
import jax.numpy as jnp
from jax import lax
from jax.experimental import pallas as pl
from jax.experimental.pallas import tpu as pltpu
from jax.experimental.pallas import tpu_sc as plsc

N_USERS = 5000
N_ITEMS = 5000
N = N_USERS + N_ITEMS
D = 256
E = 160000
NUM_LAYERS = 3

NC = 2
NS = 16
LANES = 16
HCOL = D // NC

NP_ = 10112
ROWS_PER_TILE = NP_ // NS
E_PER_TILE = E // NS
CHUNK = 80
N_CHUNKS = E_PER_TILE // CHUNK


def _spmm_body(x_hbm, src_hbm, dst_hbm, w_hbm, z_hbm, y_hbm,
               acc, rows0, rows1, src_i0, src_i1, gsrc0, gsrc1,
               dst_i0, dst_i1, w_v0, w_v1, sem0, sem1, sem_s0, sem_s1):
    c = lax.axis_index("c")
    s = lax.axis_index("s")
    rows = (rows0, rows1)
    src_i = (src_i0, src_i1)
    gsrc = (gsrc0, gsrc1)
    dst_i = (dst_i0, dst_i1)
    w_v = (w_v0, w_v1)
    sem = (sem0, sem1)

    r0 = pl.multiple_of(s * ROWS_PER_TILE, 8)
    pltpu.sync_copy(z_hbm.at[pl.ds(r0, ROWS_PER_TILE)],
                    acc.at[pl.ds(r0, ROWS_PER_TILE)])
    plsc.subcore_barrier()

    e_base = s * E_PER_TILE

    def prep(k, b):
        e0 = pl.multiple_of(e_base + k * CHUNK, 8)
        pltpu.sync_copy(src_hbm.at[pl.ds(e0, CHUNK)], src_i[b])
        pltpu.sync_copy(dst_hbm.at[pl.ds(e0, CHUNK)], dst_i[b])
        pltpu.sync_copy(w_hbm.at[pl.ds(e0, CHUNK)], w_v[b])
        for g in range(CHUNK // LANES):
            sv = src_i[b][pl.ds(g * LANES, LANES)]
            gsrc[b][pl.ds(g * LANES, LANES)] = sv + c * NP_
        return pltpu.async_copy(x_hbm.at[gsrc[b]], rows[b], sem[b])

    def scale(b):
        def scale_edge(i, carry2):
            e0 = 2 * i
            e1 = 2 * i + 1
            bw0 = plsc.load_gather(w_v[b], [jnp.full((LANES,), e0, jnp.int32)])
            bw1 = plsc.load_gather(w_v[b], [jnp.full((LANES,), e1, jnp.int32)])
            for j in range(HCOL // LANES):
                rows[b][e0, pl.ds(j * LANES, LANES)] = (
                    rows[b][e0, pl.ds(j * LANES, LANES)] * bw0)
            for j in range(HCOL // LANES):
                rows[b][e1, pl.ds(j * LANES, LANES)] = (
                    rows[b][e1, pl.ds(j * LANES, LANES)] * bw1)
            return carry2

        lax.fori_loop(0, CHUNK // 2, scale_edge, 0)

    def wait_g(b):
        pltpu.make_async_copy(x_hbm.at[gsrc[b]], rows[b], sem[b]).wait()

    prep(0, 0)

    def pair_body(p, carry):
        k = 2 * p
        prep(k + 1, 1)
        wait_g(0)
        scale(0)
        h0 = pltpu.async_copy(rows[0], acc.at[dst_i[0]], sem_s0, add=True)
        wait_g(1)
        scale(1)
        h1 = pltpu.async_copy(rows[1], acc.at[dst_i[1]], sem_s1, add=True)
        h0.wait()
        prep(k + 2, 0)
        h1.wait()
        return carry

    lax.fori_loop(0, (N_CHUNKS - 1) // 2, pair_body, 0)
    wait_g(0)
    scale(0)
    pltpu.sync_copy(rows[0], acc.at[dst_i[0]], add=True)
    plsc.subcore_barrier()

    y0 = pl.multiple_of(c * NP_ + s * ROWS_PER_TILE, 8)
    pltpu.sync_copy(acc.at[pl.ds(r0, ROWS_PER_TILE)],
                    y_hbm.at[pl.ds(y0, ROWS_PER_TILE)])


_spmm_call = functools.partial(
    pl.kernel,
    out_type=jax.ShapeDtypeStruct((NC * NP_, HCOL), jnp.float32),
    mesh=plsc.VectorSubcoreMesh(core_axis_name="c", subcore_axis_name="s"),
    compiler_params=pltpu.CompilerParams(needs_layout_passes=False),
    scratch_types=[
        pltpu.VMEM_SHARED((NP_, HCOL), jnp.float32),
        pltpu.VMEM((CHUNK, HCOL), jnp.float32),
        pltpu.VMEM((CHUNK, HCOL), jnp.float32),
        pltpu.VMEM((CHUNK,), jnp.int32),
        pltpu.VMEM((CHUNK,), jnp.int32),
        pltpu.VMEM((CHUNK,), jnp.int32),
        pltpu.VMEM((CHUNK,), jnp.int32),
        pltpu.VMEM((CHUNK,), jnp.int32),
        pltpu.VMEM((CHUNK,), jnp.int32),
        pltpu.VMEM((CHUNK,), jnp.float32),
        pltpu.VMEM((CHUNK,), jnp.float32),
        pltpu.SemaphoreType.DMA,
        pltpu.SemaphoreType.DMA,
        pltpu.SemaphoreType.DMA,
        pltpu.SemaphoreType.DMA,
    ],
)(_spmm_body)


def _to_half(x):
    pad = NP_ - N
    return jnp.concatenate(
        [jnp.pad(x[:, :HCOL], ((0, pad), (0, 0))),
         jnp.pad(x[:, HCOL:], ((0, pad), (0, 0)))], axis=0)


def _from_half(h):
    return jnp.concatenate([h[:N], h[NP_:NP_ + N]], axis=1)


LAMBDAS = [float(l + 1) / (NUM_LAYERS + 1) for l in range(NUM_LAYERS + 1)]
BIN_BLOCK = 1000


def _binarize_body(x0, x1, x2, x3, o):
    for i, (xr, lam) in enumerate(zip((x0, x1, x2, x3), LAMBDAS)):
        v = xr[...]
        m = jnp.mean(jnp.abs(v), axis=1, keepdims=True)
        o[:, i, :] = jnp.sign(v) * (m * lam)


def _binarize4(xs):
    in_spec = pl.BlockSpec((BIN_BLOCK, D), lambda i: (i, 0))
    return pl.pallas_call(
        _binarize_body,
        grid=(N // BIN_BLOCK,),
        in_specs=[in_spec] * 4,
        out_specs=pl.BlockSpec((BIN_BLOCK, NUM_LAYERS + 1, D),
                               lambda i: (i, 0, 0)),
        out_shape=jax.ShapeDtypeStruct((N, NUM_LAYERS + 1, D), jnp.float32),
    )(*xs)


def kernel(user_embed, item_embed, edge_index, edge_weight):
    con = jnp.concatenate([user_embed, item_embed], axis=0)
    src = edge_index[0]
    dst = edge_index[1]
    zeros = jnp.zeros((NP_, HCOL), jnp.float32)
    xs = [con]
    h = _to_half(con)
    for _ in range(NUM_LAYERS):
        h = _spmm_call(h, src, dst, edge_weight, zeros)
        xs.append(_from_half(h))
    return _binarize4(xs)


# speedup vs baseline: 3.8936x; 1.0187x over previous
import functools

import jax
import jax.numpy as jnp
from jax import lax
from jax.experimental import pallas as pl
from jax.experimental.pallas import tpu as pltpu
from jax.experimental.pallas import tpu_sc as plsc

N_USERS = 5000
N_ITEMS = 5000
N = N_USERS + N_ITEMS
D = 256
E = 160000
NUM_LAYERS = 3

NC = 2
NS = 16
LANES = 16
HCOL = D // NC

NP_ = 10112
ROWS_PER_TILE = NP_ // NS
E_PER_TILE = E // NS
CHUNK = 80
N_CHUNKS = E_PER_TILE // CHUNK


def _spmm_body(x_hbm, src_hbm, dst_hbm, w_hbm, z_hbm, y_hbm,
               acc, rows0, rows1, src_i0, src_i1, gsrc0, gsrc1,
               dst_i0, dst_i1, w_v0, w_v1, sem0, sem1, sem_s0, sem_s1):
    c = lax.axis_index("c")
    s = lax.axis_index("s")
    rows = (rows0, rows1)
    src_i = (src_i0, src_i1)
    gsrc = (gsrc0, gsrc1)
    dst_i = (dst_i0, dst_i1)
    w_v = (w_v0, w_v1)
    sem = (sem0, sem1)

    r0 = pl.multiple_of(s * ROWS_PER_TILE, 8)
    pltpu.sync_copy(z_hbm.at[pl.ds(r0, ROWS_PER_TILE)],
                    acc.at[pl.ds(r0, ROWS_PER_TILE)])
    plsc.subcore_barrier()

    e_base = s * E_PER_TILE

    def prep(k, b):
        e0 = pl.multiple_of(e_base + k * CHUNK, 8)
        pltpu.sync_copy(src_hbm.at[pl.ds(e0, CHUNK)], src_i[b])
        pltpu.sync_copy(dst_hbm.at[pl.ds(e0, CHUNK)], dst_i[b])
        pltpu.sync_copy(w_hbm.at[pl.ds(e0, CHUNK)], w_v[b])
        for g in range(CHUNK // LANES):
            sv = src_i[b][pl.ds(g * LANES, LANES)]
            gsrc[b][pl.ds(g * LANES, LANES)] = sv + c * NP_
        return pltpu.async_copy(x_hbm.at[gsrc[b]], rows[b], sem[b])

    def scale(b):
        def scale_edge(i, carry2):
            es = [4 * i, 4 * i + 1, 4 * i + 2, 4 * i + 3]
            bws = [plsc.load_gather(w_v[b], [jnp.full((LANES,), e, jnp.int32)])
                   for e in es]
            for e, bw in zip(es, bws):
                for j in range(HCOL // LANES):
                    rows[b][e, pl.ds(j * LANES, LANES)] = (
                        rows[b][e, pl.ds(j * LANES, LANES)] * bw)
            return carry2

        lax.fori_loop(0, CHUNK // 4, scale_edge, 0)

    def wait_g(b):
        pltpu.make_async_copy(x_hbm.at[gsrc[b]], rows[b], sem[b]).wait()

    prep(0, 0)

    def pair_body(p, carry):
        k = 2 * p
        prep(k + 1, 1)
        wait_g(0)
        scale(0)
        h0 = pltpu.async_copy(rows[0], acc.at[dst_i[0]], sem_s0, add=True)
        wait_g(1)
        scale(1)
        h1 = pltpu.async_copy(rows[1], acc.at[dst_i[1]], sem_s1, add=True)
        h0.wait()
        prep(k + 2, 0)
        h1.wait()
        return carry

    lax.fori_loop(0, (N_CHUNKS - 1) // 2, pair_body, 0)
    wait_g(0)
    scale(0)
    pltpu.sync_copy(rows[0], acc.at[dst_i[0]], add=True)
    plsc.subcore_barrier()

    y0 = pl.multiple_of(c * NP_ + s * ROWS_PER_TILE, 8)
    pltpu.sync_copy(acc.at[pl.ds(r0, ROWS_PER_TILE)],
                    y_hbm.at[pl.ds(y0, ROWS_PER_TILE)])


_spmm_call = functools.partial(
    pl.kernel,
    out_type=jax.ShapeDtypeStruct((NC * NP_, HCOL), jnp.float32),
    mesh=plsc.VectorSubcoreMesh(core_axis_name="c", subcore_axis_name="s"),
    compiler_params=pltpu.CompilerParams(needs_layout_passes=False),
    scratch_types=[
        pltpu.VMEM_SHARED((NP_, HCOL), jnp.float32),
        pltpu.VMEM((CHUNK, HCOL), jnp.float32),
        pltpu.VMEM((CHUNK, HCOL), jnp.float32),
        pltpu.VMEM((CHUNK,), jnp.int32),
        pltpu.VMEM((CHUNK,), jnp.int32),
        pltpu.VMEM((CHUNK,), jnp.int32),
        pltpu.VMEM((CHUNK,), jnp.int32),
        pltpu.VMEM((CHUNK,), jnp.int32),
        pltpu.VMEM((CHUNK,), jnp.int32),
        pltpu.VMEM((CHUNK,), jnp.float32),
        pltpu.VMEM((CHUNK,), jnp.float32),
        pltpu.SemaphoreType.DMA,
        pltpu.SemaphoreType.DMA,
        pltpu.SemaphoreType.DMA,
        pltpu.SemaphoreType.DMA,
    ],
)(_spmm_body)


def _to_half(x):
    pad = NP_ - N
    return jnp.concatenate(
        [jnp.pad(x[:, :HCOL], ((0, pad), (0, 0))),
         jnp.pad(x[:, HCOL:], ((0, pad), (0, 0)))], axis=0)


def _from_half(h):
    return jnp.concatenate([h[:N], h[NP_:NP_ + N]], axis=1)


LAMBDAS = [float(l + 1) / (NUM_LAYERS + 1) for l in range(NUM_LAYERS + 1)]
BIN_BLOCK = 1000


def _binarize_body(x0, x1, x2, x3, o):
    for i, (xr, lam) in enumerate(zip((x0, x1, x2, x3), LAMBDAS)):
        v = xr[...]
        m = jnp.mean(jnp.abs(v), axis=1, keepdims=True)
        o[:, i, :] = jnp.sign(v) * (m * lam)


def _binarize4(xs):
    in_spec = pl.BlockSpec((BIN_BLOCK, D), lambda i: (i, 0))
    return pl.pallas_call(
        _binarize_body,
        grid=(N // BIN_BLOCK,),
        in_specs=[in_spec] * 4,
        out_specs=pl.BlockSpec((BIN_BLOCK, NUM_LAYERS + 1, D),
                               lambda i: (i, 0, 0)),
        out_shape=jax.ShapeDtypeStruct((N, NUM_LAYERS + 1, D), jnp.float32),
    )(*xs)


def kernel(user_embed, item_embed, edge_index, edge_weight):
    con = jnp.concatenate([user_embed, item_embed], axis=0)
    src = edge_index[0]
    dst = edge_index[1]
    zeros = jnp.zeros((NP_, HCOL), jnp.float32)
    xs = [con]
    h = _to_half(con)
    for _ in range(NUM_LAYERS):
        h = _spmm_call(h, src, dst, edge_weight, zeros)
        xs.append(_from_half(h))
    return _binarize4(xs)
